# baseline (speedup 1.0000x reference)
import functools

import jax
import jax.numpy as jnp
from jax import lax
from jax.experimental import pallas as pl
from jax.experimental.pallas import tpu as pltpu
from jax.experimental.pallas import tpu_sc as plsc

_B = 16384
_F = 26
_D = 64
_H1 = 128
_H2 = 64
_V1 = 1000001
_W = 128

_NC = 2
_NS = 16
_NW = _NC * _NS
_BPW = _B // _NW
_CH = 8
_IDX = _CH * _F
_HIDX = _IDX // 2
_NCHUNK = _BPW // _CH
_IROWS = _BPW * _F // _HIDX
_LANES = 16

_FBLK = 16384


def _tc_fuse(embt_ref, lin_ref, eye_ref, out_ref):
    wt = lax.dot_general(embt_ref[...], eye_ref[...], (((0,), (0,)), ((), ())),
                         preferred_element_type=jnp.float32)
    out_ref[:, 0:_D] = wt
    out_ref[:, _D:_D + 1] = lin_ref[...]
    out_ref[:, _D + 1:] = jnp.zeros((_FBLK, _W - _D - 1), jnp.float32)


def _build_fused_table(emb_table, lin_table):
    embt = emb_table.T
    eye = jnp.eye(_D, dtype=jnp.float32)
    grid = (_V1 + _FBLK - 1) // _FBLK
    return pl.pallas_call(
        _tc_fuse,
        grid=(grid,),
        compiler_params=pltpu.CompilerParams(
            vmem_limit_bytes=100 * 1024 * 1024),
        in_specs=[
            pl.BlockSpec((_D, _FBLK), lambda i: (0, i)),
            pl.BlockSpec((_FBLK, 1), lambda i: (i, 0)),
            pl.BlockSpec((_D, _D), lambda i: (0, 0)),
        ],
        out_specs=pl.BlockSpec((_FBLK, _W), lambda i: (i, 0)),
        out_shape=jax.ShapeDtypeStruct((_V1, _W), jnp.float32),
    )(embt, lin_table, eye)


def _sc_gather_reduce():
    mesh = plsc.VectorSubcoreMesh(
        core_axis_name="c", subcore_axis_name="s",
        num_cores=_NC, num_subcores=_NS)

    @functools.partial(
        pl.kernel,
        out_type=(
            jax.ShapeDtypeStruct((_B, _D), jnp.float32),
            jax.ShapeDtypeStruct((_B, _D), jnp.float32),
            jax.ShapeDtypeStruct((_B * _F,), jnp.float32),
        ),
        mesh=mesh,
        compiler_params=pltpu.CompilerParams(
            use_tc_tiling_on_sc=False, needs_layout_passes=False),
        scratch_types=[
            pltpu.VMEM((_IROWS, _HIDX), jnp.int32),
            pltpu.VMEM((2, _IDX, _W), jnp.float32),
            pltpu.VMEM((2, _IDX), jnp.float32),
            pltpu.VMEM((2, _CH, _D), jnp.float32),
            pltpu.VMEM((2, _CH, _D), jnp.float32),
            pltpu.SemaphoreType.DMA,
            pltpu.SemaphoreType.DMA,
            pltpu.SemaphoreType.DMA,
        ],
    )
    def sc_fn(xidx_hbm, tab_hbm, s_hbm, q_hbm, linraw_hbm,
              idx_all, rows_v, linval_v, acc_s, acc_q, semg0, semg1, semo):
        wid = lax.axis_index("s") * _NC + lax.axis_index("c")
        row0 = wid * _BPW
        semg = (semg0, semg1)

        pltpu.sync_copy(xidx_hbm.at[pl.ds(wid * _IROWS, _IROWS)], idx_all)

        def fire(ci, b):
            for h in range(2):
                pltpu.async_copy(
                    tab_hbm.at[idx_all.at[2 * ci + h]],
                    rows_v.at[b, pl.ds(h * _HIDX, _HIDX)], semg[b])

        def drain(ci, b):
            for h in range(2):
                pltpu.make_async_copy(
                    tab_hbm.at[idx_all.at[2 * ci + h]],
                    rows_v.at[b, pl.ds(h * _HIDX, _HIDX)], semg[b]).wait()

        def out_descs(ci, o):
            out_r = row0 + ci * _CH
            return (
                (acc_s.at[o], s_hbm.at[pl.ds(out_r, _CH)]),
                (acc_q.at[o], q_hbm.at[pl.ds(out_r, _CH)]),
                (linval_v.at[o], linraw_hbm.at[pl.ds(out_r * _F, _IDX)]),
            )

        def flush_wait(ci, o):
            for src, dst in out_descs(ci, o):
                pltpu.make_async_copy(src, dst, semo).wait()

        def reduce_chunk(ci, b):
            for r in range(_CH):
                for db in range(_D // _LANES):
                    sl = pl.ds(db * _LANES, _LANES)
                    v = rows_v[b, r * _F, sl]
                    acc = v
                    accq = v * v
                    for f in range(1, _F):
                        v = rows_v[b, r * _F + f, sl]
                        acc = acc + v
                        accq = accq + v * v
                    acc_s[b, r, sl] = acc
                    acc_q[b, r, sl] = accq
            lane_iota = lax.iota(jnp.int32, _LANES)
            lane_d = jnp.full((_LANES,), _D, jnp.int32)
            for g in range(_IDX // _LANES):
                sl = pl.ds(g * _LANES, _LANES)
                rows16 = lane_iota + jnp.int32(g * _LANES)
                linval_v[b, sl] = plsc.load_gather(
                    rows_v.at[b], [rows16, lane_d])
            for src, dst in out_descs(ci, b):
                pltpu.async_copy(src, dst, semo)

        fire(0, 0)

        def outer(k, carry):
            for b in range(2):
                ci = 2 * k + b

                @pl.when(ci + 1 < _NCHUNK)
                def _():
                    fire(ci + 1, 1 - b)

                drain(ci, b)

                @pl.when(ci >= 2)
                def _():
                    flush_wait(ci - 2, b)

                reduce_chunk(ci, b)
            return carry

        lax.fori_loop(0, _NCHUNK // 2, outer, 0)
        flush_wait(_NCHUNK - 2, 0)
        flush_wait(_NCHUNK - 1, 1)

    return sc_fn


_TC_BLK = 2048


def _tc_head(s_ref, q_ref, linr_ref, w1_ref, b1_ref, w2_ref, b2_ref,
             wht_ref, c0_ref, o_ref):
    inter = 0.5 * (s_ref[...] * s_ref[...] - q_ref[...])
    h = jnp.maximum(
        lax.dot_general(inter, w1_ref[...], (((1,), (0,)), ((), ())),
                        preferred_element_type=jnp.float32) + b1_ref[...], 0.0)
    h = jnp.maximum(
        lax.dot_general(h, w2_ref[...], (((1,), (0,)), ((), ())),
                        preferred_element_type=jnp.float32) + b2_ref[...], 0.0)
    head = jnp.sum(h * wht_ref[...], axis=1)
    lin = jnp.sum(linr_ref[...], axis=1)
    out = head + lin + c0_ref[0, 0]
    o_ref[...] = jnp.clip(out, -2.0, 2.0)


def kernel(x, emb_table, lin_table, bias, W1, b1, W2, b2, Wh, bh):
    xidx = x.reshape(_B * _F).astype(jnp.int32).reshape(_NW * _IROWS, _HIDX)
    tab = _build_fused_table(emb_table, lin_table)
    s, q, linraw = _sc_gather_reduce()(xidx, tab)
    linr = linraw.reshape(_B, _F)
    c0 = (bias + bh).reshape(1, 1)
    wht = Wh.reshape(1, _H2)

    grid = _B // _TC_BLK
    out = pl.pallas_call(
        _tc_head,
        grid=(grid,),
        in_specs=[
            pl.BlockSpec((_TC_BLK, _D), lambda i: (i, 0)),
            pl.BlockSpec((_TC_BLK, _D), lambda i: (i, 0)),
            pl.BlockSpec((_TC_BLK, _F), lambda i: (i, 0)),
            pl.BlockSpec((_D, _H1), lambda i: (0, 0)),
            pl.BlockSpec((_H1,), lambda i: (0,)),
            pl.BlockSpec((_H1, _H2), lambda i: (0, 0)),
            pl.BlockSpec((_H2,), lambda i: (0,)),
            pl.BlockSpec((1, _H2), lambda i: (0, 0)),
            pl.BlockSpec((1, 1), lambda i: (0, 0)),
        ],
        out_specs=pl.BlockSpec((_TC_BLK,), lambda i: (i,)),
        out_shape=jax.ShapeDtypeStruct((_B,), jnp.float32),
    )(s, q, linr, W1, b1, W2, b2, wht, c0)
    return out


# speedup vs baseline: 1.3614x; 1.3614x over previous
import functools

import jax
import jax.numpy as jnp
from jax import lax
from jax.experimental import pallas as pl
from jax.experimental.pallas import tpu as pltpu
from jax.experimental.pallas import tpu_sc as plsc

_B = 16384
_F = 26
_D = 64
_H1 = 128
_H2 = 64
_V1 = 1000001
_W = 128

_NC = 2
_NS = 16
_NW = _NC * _NS
_BPW = _B // _NW
_CH = 8
_IDX = _CH * _F
_HIDX = _IDX // 2
_NCHUNK = _BPW // _CH
_IROWS = _BPW * _F // _HIDX
_LANES = 16

_FBLK = 16384


def _tc_fuse(embt_ref, lint_ref, eye_ref, one_ref, out_ref):
    wt = lax.dot_general(embt_ref[...], eye_ref[...], (((0,), (0,)), ((), ())),
                         preferred_element_type=jnp.float32)
    lt = lax.dot_general(lint_ref[...], one_ref[...], (((0,), (0,)), ((), ())),
                         preferred_element_type=jnp.float32)
    out_ref[:, 0:_D] = wt
    out_ref[:, _D:_D + 1] = lt
    out_ref[:, _D + 1:] = jnp.zeros((_FBLK, _W - _D - 1), jnp.float32)


def _build_fused_table(emb_table, lin_table):
    embt = emb_table.T
    lint = lin_table.T
    eye = jnp.eye(_D, dtype=jnp.float32)
    one = jnp.ones((1, 1), jnp.float32)
    grid = (_V1 + _FBLK - 1) // _FBLK
    return pl.pallas_call(
        _tc_fuse,
        grid=(grid,),
        compiler_params=pltpu.CompilerParams(
            vmem_limit_bytes=100 * 1024 * 1024),
        in_specs=[
            pl.BlockSpec((_D, _FBLK), lambda i: (0, i)),
            pl.BlockSpec((1, _FBLK), lambda i: (0, i)),
            pl.BlockSpec((_D, _D), lambda i: (0, 0)),
            pl.BlockSpec((1, 1), lambda i: (0, 0)),
        ],
        out_specs=pl.BlockSpec((_FBLK, _W), lambda i: (i, 0)),
        out_shape=jax.ShapeDtypeStruct((_V1, _W), jnp.float32),
    )(embt, lint, eye, one)


def _sc_gather_reduce():
    mesh = plsc.VectorSubcoreMesh(
        core_axis_name="c", subcore_axis_name="s",
        num_cores=_NC, num_subcores=_NS)

    @functools.partial(
        pl.kernel,
        out_type=(
            jax.ShapeDtypeStruct((_B, _D), jnp.float32),
            jax.ShapeDtypeStruct((_B, _D), jnp.float32),
            jax.ShapeDtypeStruct((_B * _F,), jnp.float32),
        ),
        mesh=mesh,
        compiler_params=pltpu.CompilerParams(
            use_tc_tiling_on_sc=False, needs_layout_passes=False),
        scratch_types=[
            pltpu.VMEM((_IROWS, _HIDX), jnp.int32),
            pltpu.VMEM((2, _IDX, _W), jnp.float32),
            pltpu.VMEM((2, _IDX), jnp.float32),
            pltpu.VMEM((2, _CH, _D), jnp.float32),
            pltpu.VMEM((2, _CH, _D), jnp.float32),
            pltpu.SemaphoreType.DMA,
            pltpu.SemaphoreType.DMA,
            pltpu.SemaphoreType.DMA,
        ],
    )
    def sc_fn(xidx_hbm, tab_hbm, s_hbm, q_hbm, linraw_hbm,
              idx_all, rows_v, linval_v, acc_s, acc_q, semg0, semg1, semo):
        wid = lax.axis_index("s") * _NC + lax.axis_index("c")
        row0 = wid * _BPW
        semg = (semg0, semg1)

        pltpu.sync_copy(xidx_hbm.at[pl.ds(wid * _IROWS, _IROWS)], idx_all)

        def fire(ci, b):
            for h in range(2):
                pltpu.async_copy(
                    tab_hbm.at[idx_all.at[2 * ci + h]],
                    rows_v.at[b, pl.ds(h * _HIDX, _HIDX)], semg[b])

        def drain(ci, b):
            for h in range(2):
                pltpu.make_async_copy(
                    tab_hbm.at[idx_all.at[2 * ci + h]],
                    rows_v.at[b, pl.ds(h * _HIDX, _HIDX)], semg[b]).wait()

        def out_descs(ci, o):
            out_r = row0 + ci * _CH
            return (
                (acc_s.at[o], s_hbm.at[pl.ds(out_r, _CH)]),
                (acc_q.at[o], q_hbm.at[pl.ds(out_r, _CH)]),
                (linval_v.at[o], linraw_hbm.at[pl.ds(out_r * _F, _IDX)]),
            )

        def flush_wait(ci, o):
            for src, dst in out_descs(ci, o):
                pltpu.make_async_copy(src, dst, semo).wait()

        def reduce_chunk(ci, b):
            for r in range(_CH):
                for db in range(_D // _LANES):
                    sl = pl.ds(db * _LANES, _LANES)
                    v = rows_v[b, r * _F, sl]
                    acc = v
                    accq = v * v
                    for f in range(1, _F):
                        v = rows_v[b, r * _F + f, sl]
                        acc = acc + v
                        accq = accq + v * v
                    acc_s[b, r, sl] = acc
                    acc_q[b, r, sl] = accq
            lane_iota = lax.iota(jnp.int32, _LANES)
            lane_d = jnp.full((_LANES,), _D, jnp.int32)
            for g in range(_IDX // _LANES):
                sl = pl.ds(g * _LANES, _LANES)
                rows16 = lane_iota + jnp.int32(g * _LANES)
                linval_v[b, sl] = plsc.load_gather(
                    rows_v.at[b], [rows16, lane_d])
            for src, dst in out_descs(ci, b):
                pltpu.async_copy(src, dst, semo)

        fire(0, 0)

        def outer(k, carry):
            for b in range(2):
                ci = 2 * k + b

                @pl.when(ci + 1 < _NCHUNK)
                def _():
                    fire(ci + 1, 1 - b)

                drain(ci, b)

                @pl.when(ci >= 2)
                def _():
                    flush_wait(ci - 2, b)

                reduce_chunk(ci, b)
            return carry

        lax.fori_loop(0, _NCHUNK // 2, outer, 0)
        flush_wait(_NCHUNK - 2, 0)
        flush_wait(_NCHUNK - 1, 1)

    return sc_fn


_TC_BLK = 2048


def _tc_head(s_ref, q_ref, linr_ref, w1_ref, b1_ref, w2_ref, b2_ref,
             wht_ref, c0_ref, o_ref):
    inter = 0.5 * (s_ref[...] * s_ref[...] - q_ref[...])
    h = jnp.maximum(
        lax.dot_general(inter, w1_ref[...], (((1,), (0,)), ((), ())),
                        preferred_element_type=jnp.float32) + b1_ref[...], 0.0)
    h = jnp.maximum(
        lax.dot_general(h, w2_ref[...], (((1,), (0,)), ((), ())),
                        preferred_element_type=jnp.float32) + b2_ref[...], 0.0)
    head = jnp.sum(h * wht_ref[...], axis=1)
    lin = jnp.sum(linr_ref[...], axis=1)
    out = head + lin + c0_ref[0, 0]
    o_ref[...] = jnp.clip(out, -2.0, 2.0)


def kernel(x, emb_table, lin_table, bias, W1, b1, W2, b2, Wh, bh):
    xidx = x.reshape(_B * _F).astype(jnp.int32).reshape(_NW * _IROWS, _HIDX)
    tab = _build_fused_table(emb_table, lin_table)
    s, q, linraw = _sc_gather_reduce()(xidx, tab)
    linr = linraw.reshape(_B, _F)
    c0 = (bias + bh).reshape(1, 1)
    wht = Wh.reshape(1, _H2)

    grid = _B // _TC_BLK
    out = pl.pallas_call(
        _tc_head,
        grid=(grid,),
        in_specs=[
            pl.BlockSpec((_TC_BLK, _D), lambda i: (i, 0)),
            pl.BlockSpec((_TC_BLK, _D), lambda i: (i, 0)),
            pl.BlockSpec((_TC_BLK, _F), lambda i: (i, 0)),
            pl.BlockSpec((_D, _H1), lambda i: (0, 0)),
            pl.BlockSpec((_H1,), lambda i: (0,)),
            pl.BlockSpec((_H1, _H2), lambda i: (0, 0)),
            pl.BlockSpec((_H2,), lambda i: (0,)),
            pl.BlockSpec((1, _H2), lambda i: (0, 0)),
            pl.BlockSpec((1, 1), lambda i: (0, 0)),
        ],
        out_specs=pl.BlockSpec((_TC_BLK,), lambda i: (i,)),
        out_shape=jax.ShapeDtypeStruct((_B,), jnp.float32),
    )(s, q, linr, W1, b1, W2, b2, wht, c0)
    return out


# speedup vs baseline: 1.5674x; 1.1513x over previous
import functools

import jax
import jax.numpy as jnp
from jax import lax
from jax.experimental import pallas as pl
from jax.experimental.pallas import tpu as pltpu
from jax.experimental.pallas import tpu_sc as plsc

_B = 16384
_F = 26
_D = 64
_H1 = 128
_H2 = 64
_V1 = 1000001
_W = 128

_NC = 2
_NS = 16
_NW = _NC * _NS
_BPW = _B // _NW
_CH = 8
_IDX = _CH * _F
_HIDX = _IDX // 2
_NCHUNK = _BPW // _CH
_IROWS = _BPW * _F // _HIDX
_LANES = 16

_FBLK = 16384


def _tc_fuse(embt_ref, lin_ref, eye_ref, out_ref):
    wt = lax.dot_general(embt_ref[...], eye_ref[...], (((0,), (0,)), ((), ())),
                         preferred_element_type=jnp.float32)
    out_ref[:, 0:_D] = wt
    out_ref[:, _D:_D + 1] = lin_ref[...].reshape(_FBLK, 1)
    out_ref[:, _D + 1:] = jnp.zeros((_FBLK, _W - _D - 1), jnp.float32)


def _build_fused_table(emb_table, lin_table):
    embt = emb_table.T
    lin_flat = lin_table.reshape(_V1)
    eye = jnp.eye(_D, dtype=jnp.float32)
    grid = (_V1 + _FBLK - 1) // _FBLK
    return pl.pallas_call(
        _tc_fuse,
        grid=(grid,),
        compiler_params=pltpu.CompilerParams(
            vmem_limit_bytes=100 * 1024 * 1024),
        in_specs=[
            pl.BlockSpec((_D, _FBLK), lambda i: (0, i)),
            pl.BlockSpec((_FBLK,), lambda i: (i,)),
            pl.BlockSpec((_D, _D), lambda i: (0, 0)),
        ],
        out_specs=pl.BlockSpec((_FBLK, _W), lambda i: (i, 0)),
        out_shape=jax.ShapeDtypeStruct((_V1, _W), jnp.float32),
    )(embt, lin_flat, eye)


def _sc_gather_reduce():
    mesh = plsc.VectorSubcoreMesh(
        core_axis_name="c", subcore_axis_name="s",
        num_cores=_NC, num_subcores=_NS)

    @functools.partial(
        pl.kernel,
        out_type=(
            jax.ShapeDtypeStruct((_B, _D), jnp.float32),
            jax.ShapeDtypeStruct((_B, _D), jnp.float32),
            jax.ShapeDtypeStruct((_B * _F,), jnp.float32),
        ),
        mesh=mesh,
        compiler_params=pltpu.CompilerParams(
            use_tc_tiling_on_sc=False, needs_layout_passes=False),
        scratch_types=[
            pltpu.VMEM((_IROWS, _HIDX), jnp.int32),
            pltpu.VMEM((2, _IDX, _W), jnp.float32),
            pltpu.VMEM((2, _IDX), jnp.float32),
            pltpu.VMEM((2, _CH, _D), jnp.float32),
            pltpu.VMEM((2, _CH, _D), jnp.float32),
            pltpu.SemaphoreType.DMA,
            pltpu.SemaphoreType.DMA,
            pltpu.SemaphoreType.DMA,
        ],
    )
    def sc_fn(xidx_hbm, tab_hbm, s_hbm, q_hbm, linraw_hbm,
              idx_all, rows_v, linval_v, acc_s, acc_q, semg0, semg1, semo):
        wid = lax.axis_index("s") * _NC + lax.axis_index("c")
        row0 = wid * _BPW
        semg = (semg0, semg1)

        pltpu.sync_copy(xidx_hbm.at[pl.ds(wid * _IROWS, _IROWS)], idx_all)

        def fire(ci, b):
            for h in range(2):
                pltpu.async_copy(
                    tab_hbm.at[idx_all.at[2 * ci + h]],
                    rows_v.at[b, pl.ds(h * _HIDX, _HIDX)], semg[b])

        def drain(ci, b):
            for h in range(2):
                pltpu.make_async_copy(
                    tab_hbm.at[idx_all.at[2 * ci + h]],
                    rows_v.at[b, pl.ds(h * _HIDX, _HIDX)], semg[b]).wait()

        def out_descs(ci, o):
            out_r = row0 + ci * _CH
            return (
                (acc_s.at[o], s_hbm.at[pl.ds(out_r, _CH)]),
                (acc_q.at[o], q_hbm.at[pl.ds(out_r, _CH)]),
                (linval_v.at[o], linraw_hbm.at[pl.ds(out_r * _F, _IDX)]),
            )

        def flush_wait(ci, o):
            for src, dst in out_descs(ci, o):
                pltpu.make_async_copy(src, dst, semo).wait()

        def reduce_chunk(ci, b):
            for r in range(_CH):
                for db in range(_D // _LANES):
                    sl = pl.ds(db * _LANES, _LANES)
                    v = rows_v[b, r * _F, sl]
                    acc = v
                    accq = v * v
                    for f in range(1, _F):
                        v = rows_v[b, r * _F + f, sl]
                        acc = acc + v
                        accq = accq + v * v
                    acc_s[b, r, sl] = acc
                    acc_q[b, r, sl] = accq
            lane_iota = lax.iota(jnp.int32, _LANES)
            lane_d = jnp.full((_LANES,), _D, jnp.int32)
            for g in range(_IDX // _LANES):
                sl = pl.ds(g * _LANES, _LANES)
                rows16 = lane_iota + jnp.int32(g * _LANES)
                linval_v[b, sl] = plsc.load_gather(
                    rows_v.at[b], [rows16, lane_d])
            for src, dst in out_descs(ci, b):
                pltpu.async_copy(src, dst, semo)

        fire(0, 0)

        def outer(k, carry):
            for b in range(2):
                ci = 2 * k + b

                @pl.when(ci + 1 < _NCHUNK)
                def _():
                    fire(ci + 1, 1 - b)

                drain(ci, b)

                @pl.when(ci >= 2)
                def _():
                    flush_wait(ci - 2, b)

                reduce_chunk(ci, b)
            return carry

        lax.fori_loop(0, _NCHUNK // 2, outer, 0)
        flush_wait(_NCHUNK - 2, 0)
        flush_wait(_NCHUNK - 1, 1)

    return sc_fn


_TC_BLK = 2048


def _tc_head(s_ref, q_ref, linr_ref, w1_ref, b1_ref, w2_ref, b2_ref,
             wht_ref, c0_ref, o_ref):
    inter = 0.5 * (s_ref[...] * s_ref[...] - q_ref[...])
    h = jnp.maximum(
        lax.dot_general(inter, w1_ref[...], (((1,), (0,)), ((), ())),
                        preferred_element_type=jnp.float32) + b1_ref[...], 0.0)
    h = jnp.maximum(
        lax.dot_general(h, w2_ref[...], (((1,), (0,)), ((), ())),
                        preferred_element_type=jnp.float32) + b2_ref[...], 0.0)
    head = jnp.sum(h * wht_ref[...], axis=1)
    lin = jnp.sum(linr_ref[...], axis=1)
    out = head + lin + c0_ref[0, 0]
    o_ref[...] = jnp.clip(out, -2.0, 2.0)


def kernel(x, emb_table, lin_table, bias, W1, b1, W2, b2, Wh, bh):
    xidx = x.reshape(_B * _F).astype(jnp.int32).reshape(_NW * _IROWS, _HIDX)
    tab = _build_fused_table(emb_table, lin_table)
    s, q, linraw = _sc_gather_reduce()(xidx, tab)
    linr = linraw.reshape(_B, _F)
    c0 = (bias + bh).reshape(1, 1)
    wht = Wh.reshape(1, _H2)

    grid = _B // _TC_BLK
    out = pl.pallas_call(
        _tc_head,
        grid=(grid,),
        in_specs=[
            pl.BlockSpec((_TC_BLK, _D), lambda i: (i, 0)),
            pl.BlockSpec((_TC_BLK, _D), lambda i: (i, 0)),
            pl.BlockSpec((_TC_BLK, _F), lambda i: (i, 0)),
            pl.BlockSpec((_D, _H1), lambda i: (0, 0)),
            pl.BlockSpec((_H1,), lambda i: (0,)),
            pl.BlockSpec((_H1, _H2), lambda i: (0, 0)),
            pl.BlockSpec((_H2,), lambda i: (0,)),
            pl.BlockSpec((1, _H2), lambda i: (0, 0)),
            pl.BlockSpec((1, 1), lambda i: (0, 0)),
        ],
        out_specs=pl.BlockSpec((_TC_BLK,), lambda i: (i,)),
        out_shape=jax.ShapeDtypeStruct((_B,), jnp.float32),
    )(s, q, linr, W1, b1, W2, b2, wht, c0)
    return out


# speedup vs baseline: 1.6386x; 1.0455x over previous
import functools

import jax
import jax.numpy as jnp
from jax import lax
from jax.experimental import pallas as pl
from jax.experimental.pallas import tpu as pltpu
from jax.experimental.pallas import tpu_sc as plsc

_B = 16384
_F = 26
_D = 64
_H1 = 128
_H2 = 64
_V1 = 1000001
_W = 128

_NC = 2
_NS = 16
_NW = _NC * _NS
_BPW = _B // _NW
_CH = 4
_IDX = _CH * _F
_NCHUNK = _BPW // _CH
_IROWS = _NCHUNK
_NBUF = 4
_LANES = 16

_FBLK = 16384


def _tc_fuse(embt_ref, lin_ref, eye_ref, out_ref):
    wt = lax.dot_general(embt_ref[...], eye_ref[...], (((0,), (0,)), ((), ())),
                         preferred_element_type=jnp.float32)
    out_ref[:, 0:_D] = wt
    out_ref[:, _D:_D + 1] = lin_ref[...].reshape(_FBLK, 1)
    out_ref[:, _D + 1:] = jnp.zeros((_FBLK, _W - _D - 1), jnp.float32)


def _build_fused_table(emb_table, lin_table):
    embt = emb_table.T
    lin_flat = lin_table.reshape(_V1)
    eye = jnp.eye(_D, dtype=jnp.float32)
    grid = (_V1 + _FBLK - 1) // _FBLK
    return pl.pallas_call(
        _tc_fuse,
        grid=(grid,),
        compiler_params=pltpu.CompilerParams(
            vmem_limit_bytes=100 * 1024 * 1024),
        in_specs=[
            pl.BlockSpec((_D, _FBLK), lambda i: (0, i)),
            pl.BlockSpec((_FBLK,), lambda i: (i,)),
            pl.BlockSpec((_D, _D), lambda i: (0, 0)),
        ],
        out_specs=pl.BlockSpec((_FBLK, _W), lambda i: (i, 0)),
        out_shape=jax.ShapeDtypeStruct((_V1, _W), jnp.float32),
    )(embt, lin_flat, eye)


def _sc_gather_reduce():
    mesh = plsc.VectorSubcoreMesh(
        core_axis_name="c", subcore_axis_name="s",
        num_cores=_NC, num_subcores=_NS)

    @functools.partial(
        pl.kernel,
        out_type=(
            jax.ShapeDtypeStruct((_B, _D), jnp.float32),
            jax.ShapeDtypeStruct((_B, _D), jnp.float32),
            jax.ShapeDtypeStruct((_B * _F,), jnp.float32),
        ),
        mesh=mesh,
        compiler_params=pltpu.CompilerParams(
            use_tc_tiling_on_sc=False, needs_layout_passes=False),
        scratch_types=[
            pltpu.VMEM((_IROWS, _IDX), jnp.int32),
            pltpu.VMEM((_NBUF, _IDX, _W), jnp.float32),
            pltpu.VMEM((_NBUF, 112), jnp.float32),
            pltpu.VMEM((_NBUF, _CH, _D), jnp.float32),
            pltpu.VMEM((_NBUF, _CH, _D), jnp.float32),
            pltpu.SemaphoreType.DMA,
            pltpu.SemaphoreType.DMA,
            pltpu.SemaphoreType.DMA,
            pltpu.SemaphoreType.DMA,
            pltpu.SemaphoreType.DMA,
        ],
    )
    def sc_fn(xidx_hbm, tab_hbm, s_hbm, q_hbm, linraw_hbm,
              idx_all, rows_v, linval_v, acc_s, acc_q,
              semg0, semg1, semg2, semg3, semo):
        wid = lax.axis_index("s") * _NC + lax.axis_index("c")
        row0 = wid * _BPW
        semg = (semg0, semg1, semg2, semg3)

        pltpu.sync_copy(xidx_hbm.at[pl.ds(wid * _IROWS, _IROWS)], idx_all)

        def fire(ci, b):
            pltpu.async_copy(
                tab_hbm.at[idx_all.at[ci]], rows_v.at[b], semg[b])

        def drain(ci, b):
            pltpu.make_async_copy(
                tab_hbm.at[idx_all.at[ci]], rows_v.at[b], semg[b]).wait()

        def out_descs(ci, o):
            out_r = row0 + ci * _CH
            return (
                (acc_s.at[o], s_hbm.at[pl.ds(out_r, _CH)]),
                (acc_q.at[o], q_hbm.at[pl.ds(out_r, _CH)]),
                (linval_v.at[o, pl.ds(0, _IDX)],
                 linraw_hbm.at[pl.ds(out_r * _F, _IDX)]),
            )

        def flush_wait(ci, o):
            for src, dst in out_descs(ci, o):
                pltpu.make_async_copy(src, dst, semo).wait()

        def reduce_chunk(ci, b):
            for r in range(_CH):
                for db in range(_D // _LANES):
                    sl = pl.ds(db * _LANES, _LANES)
                    v = rows_v[b, r * _F, sl]
                    acc = v
                    accq = v * v
                    for f in range(1, _F):
                        v = rows_v[b, r * _F + f, sl]
                        acc = acc + v
                        accq = accq + v * v
                    acc_s[b, r, sl] = acc
                    acc_q[b, r, sl] = accq
            lane_iota = lax.iota(jnp.int32, _LANES)
            lane_d = jnp.full((_LANES,), _D, jnp.int32)
            for g in range(7):
                sl = pl.ds(g * _LANES, _LANES)
                rows16 = jnp.minimum(lane_iota + jnp.int32(g * _LANES),
                                     jnp.int32(_IDX - 1))
                linval_v[b, sl] = plsc.load_gather(
                    rows_v.at[b], [rows16, lane_d])
            for src, dst in out_descs(ci, b):
                pltpu.async_copy(src, dst, semo)

        for j in range(_NBUF - 1):
            fire(j, j)

        def outer(k, carry):
            for b in range(_NBUF):
                ci = _NBUF * k + b

                @pl.when(ci + _NBUF - 1 < _NCHUNK)
                def _():
                    fire(ci + _NBUF - 1, (b + _NBUF - 1) % _NBUF)

                drain(ci, b)

                @pl.when(ci >= _NBUF)
                def _():
                    flush_wait(ci - _NBUF, b)

                reduce_chunk(ci, b)
            return carry

        lax.fori_loop(0, _NCHUNK // _NBUF, outer, 0)
        for j in range(_NBUF):
            flush_wait(_NCHUNK - _NBUF + j, j)

    return sc_fn


_TC_BLK = 2048


def _tc_head(s_ref, q_ref, linr_ref, w1_ref, b1_ref, w2_ref, b2_ref,
             wht_ref, c0_ref, o_ref):
    inter = 0.5 * (s_ref[...] * s_ref[...] - q_ref[...])
    h = jnp.maximum(
        lax.dot_general(inter, w1_ref[...], (((1,), (0,)), ((), ())),
                        preferred_element_type=jnp.float32) + b1_ref[...], 0.0)
    h = jnp.maximum(
        lax.dot_general(h, w2_ref[...], (((1,), (0,)), ((), ())),
                        preferred_element_type=jnp.float32) + b2_ref[...], 0.0)
    head = jnp.sum(h * wht_ref[...], axis=1)
    lin = jnp.sum(linr_ref[...], axis=1)
    out = head + lin + c0_ref[0, 0]
    o_ref[...] = jnp.clip(out, -2.0, 2.0)


def kernel(x, emb_table, lin_table, bias, W1, b1, W2, b2, Wh, bh):
    xidx = x.reshape(_B * _F).astype(jnp.int32).reshape(_NW * _IROWS, _IDX)
    tab = _build_fused_table(emb_table, lin_table)
    s, q, linraw = _sc_gather_reduce()(xidx, tab)
    linr = linraw.reshape(_B, _F)
    c0 = (bias + bh).reshape(1, 1)
    wht = Wh.reshape(1, _H2)

    grid = _B // _TC_BLK
    out = pl.pallas_call(
        _tc_head,
        grid=(grid,),
        in_specs=[
            pl.BlockSpec((_TC_BLK, _D), lambda i: (i, 0)),
            pl.BlockSpec((_TC_BLK, _D), lambda i: (i, 0)),
            pl.BlockSpec((_TC_BLK, _F), lambda i: (i, 0)),
            pl.BlockSpec((_D, _H1), lambda i: (0, 0)),
            pl.BlockSpec((_H1,), lambda i: (0,)),
            pl.BlockSpec((_H1, _H2), lambda i: (0, 0)),
            pl.BlockSpec((_H2,), lambda i: (0,)),
            pl.BlockSpec((1, _H2), lambda i: (0, 0)),
            pl.BlockSpec((1, 1), lambda i: (0, 0)),
        ],
        out_specs=pl.BlockSpec((_TC_BLK,), lambda i: (i,)),
        out_shape=jax.ShapeDtypeStruct((_B,), jnp.float32),
    )(s, q, linr, W1, b1, W2, b2, wht, c0)
    return out
